# Initial kernel scaffold; baseline (speedup 1.0000x reference)
import jax
import jax.numpy as jnp
from jax.experimental import pallas as pl


def kernel(prev_kc, curr_kc, prev_corr, A, W):
    raise NotImplementedError("write your pallas kernel here")


# speedup vs baseline: 1.0896x; 1.0896x over previous
import functools

import jax
import jax.numpy as jnp
from jax import lax
from jax.experimental import pallas as pl
from jax.experimental.pallas import tpu as pltpu
from jax.experimental.pallas import tpu_sc as plsc

KC_PAD = 64
W_PAD = 8
L = 16


def _kc_of_body(A_ref, out_ref):
    a = A_ref[...]
    ramp = lax.broadcasted_iota(jnp.int32, a.shape, 1).astype(jnp.float32)
    out_ref[...] = jnp.sum(a * ramp, axis=1).astype(jnp.int32)


def _compute_kc_of(Apad):
    V = Apad.shape[0]
    return pl.pallas_call(
        _kc_of_body,
        out_shape=jax.ShapeDtypeStruct((V,), jnp.int32),
    )(Apad)


def _sc_gather_ids(table, idx):
    B = idx.shape[0]
    V = table.shape[0]
    info = plsc.get_sparse_core_info()
    nw = info.num_cores * info.num_subcores
    b_per_w = B // nw
    mesh = plsc.VectorSubcoreMesh(core_axis_name="c", subcore_axis_name="s")

    @functools.partial(
        pl.kernel,
        mesh=mesh,
        out_type=jax.ShapeDtypeStruct((B,), jnp.int32),
        compiler_params=pltpu.CompilerParams(needs_layout_passes=False),
        scratch_types=[
            pltpu.VMEM((V,), jnp.int32),
            pltpu.VMEM((b_per_w,), jnp.int32),
            pltpu.VMEM((b_per_w,), jnp.int32),
        ],
    )
    def k(table_hbm, idx_hbm, out_hbm, table_v, idx_v, out_v):
        wid = lax.axis_index("s") * info.num_cores + lax.axis_index("c")
        base = wid * b_per_w
        pltpu.sync_copy(table_hbm, table_v)
        pltpu.sync_copy(idx_hbm.at[pl.ds(base, b_per_w)], idx_v)
        for i in range(b_per_w // L):
            ivec = idx_v[pl.ds(i * L, L)]
            out_v[pl.ds(i * L, L)] = plsc.load_gather(table_v, [ivec])
        pltpu.sync_copy(out_v, out_hbm.at[pl.ds(base, b_per_w)])

    return k(table, idx)


def _bkt_body(pk_ref, ck_ref, pcor_ref, W_ref, At_ref, probs_ref, state_ref):
    blk = pk_ref.shape[0]
    T = pk_ref.shape[1]
    Wp = W_ref[...]
    init = jax.nn.sigmoid(Wp[:, 4])
    state = jnp.broadcast_to(init[None, :], (blk, KC_PAD))
    ramp = lax.broadcasted_iota(jnp.int32, (blk, KC_PAD), 1)
    pcs = []
    for t in range(T):
        cA = (ramp == ck_ref[:, t:t + 1]).astype(jnp.float32)
        cp = jax.nn.sigmoid(
            jnp.dot(cA, Wp, preferred_element_type=jnp.float32))
        if t > 0:
            pA = (ramp == pk_ref[:, t:t + 1]).astype(jnp.float32)
            pp = jax.nn.sigmoid(
                jnp.dot(pA, Wp, preferred_element_type=jnp.float32))
            pcor = pcor_ref[:, t:t + 1]
            ss = jnp.sum(state * pA, axis=1, keepdims=True)
            p2, p3 = pp[:, 2:3], pp[:, 3:4]
            po0 = jnp.where(pcor > 0.5, p2, 1.0 - p2)
            po1 = jnp.where(pcor > 0.5, p3, 1.0 - p3)
            filt = po1 * ss / (po0 * (1.0 - ss) + po1 * ss)
            pred = pp[:, 0:1] * (1.0 - filt) + (1.0 - pp[:, 1:2]) * filt
            state = state * (1.0 - pA) + pA * pred
        cs = jnp.sum(state * cA, axis=1, keepdims=True)
        pcs.append(cp[:, 2:3] * (1.0 - cs) + cp[:, 3:4] * cs)
    probs_ref[...] = jnp.concatenate(pcs, axis=1)
    state_ref[...] = jnp.dot(state, At_ref[...],
                             preferred_element_type=jnp.float32)


def kernel(prev_kc, curr_kc, prev_corr, A, W):
    B, T = prev_kc.shape
    V, K = A.shape
    Apad = jnp.pad(A, ((0, 0), (0, KC_PAD - K)))
    Wpad = jnp.pad(W, ((0, KC_PAD - K), (0, W_PAD - W.shape[1])))
    At = Apad.T

    kc_of = _compute_kc_of(Apad)
    idx = jnp.concatenate(
        [prev_kc.reshape(-1), curr_kc.reshape(-1)]).astype(jnp.int32)
    ids = _sc_gather_ids(kc_of, idx)
    pk = ids[:B * T].reshape(B, T)
    ck = ids[B * T:].reshape(B, T)

    blk = 128
    probs, state = pl.pallas_call(
        _bkt_body,
        grid=(B // blk,),
        in_specs=[
            pl.BlockSpec((blk, T), lambda i: (i, 0)),
            pl.BlockSpec((blk, T), lambda i: (i, 0)),
            pl.BlockSpec((blk, T), lambda i: (i, 0)),
            pl.BlockSpec((KC_PAD, W_PAD), lambda i: (0, 0)),
            pl.BlockSpec((KC_PAD, V), lambda i: (0, 0)),
        ],
        out_specs=[
            pl.BlockSpec((blk, T), lambda i: (i, 0)),
            pl.BlockSpec((blk, V), lambda i: (i, 0)),
        ],
        out_shape=[
            jax.ShapeDtypeStruct((B, T), jnp.float32),
            jax.ShapeDtypeStruct((B, V), jnp.float32),
        ],
    )(pk, ck, prev_corr, Wpad, At)
    return probs, state


# speedup vs baseline: 14.5145x; 13.3204x over previous
import functools

import jax
import jax.numpy as jnp
from jax import lax
from jax.experimental import pallas as pl
from jax.experimental.pallas import tpu as pltpu
from jax.experimental.pallas import tpu_sc as plsc

KC_PAD = 64
W_PAD = 8
L = 16


def _kc_of_body(A_ref, out_ref):
    a = A_ref[...]
    ramp = lax.broadcasted_iota(jnp.int32, a.shape, 1).astype(jnp.float32)
    out_ref[...] = jnp.sum(a * ramp, axis=1).astype(jnp.int32)


def _compute_kc_of(Apad):
    V = Apad.shape[0]
    return pl.pallas_call(
        _kc_of_body,
        out_shape=jax.ShapeDtypeStruct((V,), jnp.int32),
    )(Apad)


def _sc_gather_ids(table, idx):
    B = idx.shape[0]
    V = table.shape[0]
    info = plsc.get_sparse_core_info()
    nw = info.num_cores * info.num_subcores
    b_per_w = B // nw
    mesh = plsc.VectorSubcoreMesh(core_axis_name="c", subcore_axis_name="s")

    @functools.partial(
        pl.kernel,
        mesh=mesh,
        out_type=jax.ShapeDtypeStruct((B,), jnp.int32),
        compiler_params=pltpu.CompilerParams(needs_layout_passes=False),
        scratch_types=[
            pltpu.VMEM((V,), jnp.int32),
            pltpu.VMEM((b_per_w,), jnp.int32),
            pltpu.VMEM((b_per_w,), jnp.int32),
        ],
    )
    def k(table_hbm, idx_hbm, out_hbm, table_v, idx_v, out_v):
        wid = lax.axis_index("s") * info.num_cores + lax.axis_index("c")
        base = wid * b_per_w
        pltpu.sync_copy(table_hbm, table_v)
        pltpu.sync_copy(idx_hbm.at[pl.ds(base, b_per_w)], idx_v)
        for i in range(b_per_w // L):
            ivec = idx_v[pl.ds(i * L, L)]
            out_v[pl.ds(i * L, L)] = plsc.load_gather(table_v, [ivec])
        pltpu.sync_copy(out_v, out_hbm.at[pl.ds(base, b_per_w)])

    return k(table, idx)


def _bkt_body(pk_ref, ck_ref, pcor_ref, W_ref, At_ref, probs_ref, state_ref):
    T = pk_ref.shape[0]
    blk = pk_ref.shape[1]
    Wp = W_ref[...]
    ramp = lax.broadcasted_iota(jnp.int32, (KC_PAD, blk), 0)

    def wsel(oh, c):
        return jnp.sum(oh * Wp[:, c:c + 1], axis=0, keepdims=True)

    state = jnp.broadcast_to(
        jax.nn.sigmoid(Wp[:, 4:5]), (KC_PAD, blk))
    for t in range(T):
        oc = (ramp == ck_ref[t:t + 1, :]).astype(jnp.float32)
        c2 = jax.nn.sigmoid(wsel(oc, 2))
        c3 = jax.nn.sigmoid(wsel(oc, 3))
        if t > 0:
            op = (ramp == pk_ref[t:t + 1, :]).astype(jnp.float32)
            p0 = jax.nn.sigmoid(wsel(op, 0))
            p1 = jax.nn.sigmoid(wsel(op, 1))
            p2 = jax.nn.sigmoid(wsel(op, 2))
            p3 = jax.nn.sigmoid(wsel(op, 3))
            pcor = pcor_ref[t:t + 1, :]
            ss = jnp.sum(state * op, axis=0, keepdims=True)
            po0 = jnp.where(pcor > 0.5, p2, 1.0 - p2)
            po1 = jnp.where(pcor > 0.5, p3, 1.0 - p3)
            filt = po1 * ss / (po0 * (1.0 - ss) + po1 * ss)
            pred = p0 * (1.0 - filt) + (1.0 - p1) * filt
            state = state * (1.0 - op) + op * pred
        cs = jnp.sum(state * oc, axis=0, keepdims=True)
        probs_ref[t:t + 1, :] = c2 * (1.0 - cs) + c3 * cs
    state_ref[...] = jax.lax.dot_general(
        state, At_ref[...], (((0,), (0,)), ((), ())),
        preferred_element_type=jnp.float32)


def kernel(prev_kc, curr_kc, prev_corr, A, W):
    B, T = prev_kc.shape
    V, K = A.shape
    Apad = jnp.pad(A, ((0, 0), (0, KC_PAD - K)))
    Wpad = jnp.pad(W, ((0, KC_PAD - K), (0, W_PAD - W.shape[1])))
    At = Apad.T

    kc_of = _compute_kc_of(Apad)
    idx = jnp.concatenate(
        [prev_kc.reshape(-1), curr_kc.reshape(-1)]).astype(jnp.int32)
    ids = _sc_gather_ids(kc_of, idx)
    pkT = ids[:B * T].reshape(B, T).T
    ckT = ids[B * T:].reshape(B, T).T

    blk = 128
    probsT, state = pl.pallas_call(
        _bkt_body,
        grid=(B // blk,),
        in_specs=[
            pl.BlockSpec((T, blk), lambda i: (0, i)),
            pl.BlockSpec((T, blk), lambda i: (0, i)),
            pl.BlockSpec((T, blk), lambda i: (0, i)),
            pl.BlockSpec((KC_PAD, W_PAD), lambda i: (0, 0)),
            pl.BlockSpec((KC_PAD, V), lambda i: (0, 0)),
        ],
        out_specs=[
            pl.BlockSpec((T, blk), lambda i: (0, i)),
            pl.BlockSpec((blk, V), lambda i: (i, 0)),
        ],
        out_shape=[
            jax.ShapeDtypeStruct((T, B), jnp.float32),
            jax.ShapeDtypeStruct((B, V), jnp.float32),
        ],
    )(pkT, ckT, prev_corr.T, Wpad, At)
    return probsT.T, state


# speedup vs baseline: 23.0096x; 1.5853x over previous
import functools

import jax
import jax.numpy as jnp
from jax import lax
from jax.experimental import pallas as pl
from jax.experimental.pallas import tpu as pltpu
from jax.experimental.pallas import tpu_sc as plsc

KC_PAD = 64
W_PAD = 8
L = 16


def _kc_of_body(A_ref, out_ref):
    a = A_ref[...]
    ramp = lax.broadcasted_iota(jnp.int32, a.shape, 1).astype(jnp.float32)
    out_ref[...] = jnp.sum(a * ramp, axis=1).astype(jnp.int32)


def _compute_kc_of(Apad):
    V = Apad.shape[0]
    return pl.pallas_call(
        _kc_of_body,
        out_shape=jax.ShapeDtypeStruct((V,), jnp.int32),
    )(Apad)


def _sc_gather_ids(table, idx):
    B = idx.shape[0]
    V = table.shape[0]
    info = plsc.get_sparse_core_info()
    nw = info.num_cores * info.num_subcores
    b_per_w = B // nw
    mesh = plsc.VectorSubcoreMesh(core_axis_name="c", subcore_axis_name="s")

    @functools.partial(
        pl.kernel,
        mesh=mesh,
        out_type=jax.ShapeDtypeStruct((B,), jnp.int32),
        compiler_params=pltpu.CompilerParams(needs_layout_passes=False),
        scratch_types=[
            pltpu.VMEM((V,), jnp.int32),
            pltpu.VMEM((b_per_w,), jnp.int32),
            pltpu.VMEM((b_per_w,), jnp.int32),
        ],
    )
    def k(table_hbm, idx_hbm, out_hbm, table_v, idx_v, out_v):
        wid = lax.axis_index("s") * info.num_cores + lax.axis_index("c")
        base = wid * b_per_w
        pltpu.sync_copy(table_hbm, table_v)
        pltpu.sync_copy(idx_hbm.at[pl.ds(base, b_per_w)], idx_v)
        for i in range(b_per_w // L):
            ivec = idx_v[pl.ds(i * L, L)]
            out_v[pl.ds(i * L, L)] = plsc.load_gather(table_v, [ivec])
        pltpu.sync_copy(out_v, out_hbm.at[pl.ds(base, b_per_w)])

    return k(table, idx)


def _bkt_body(pk_ref, ck_ref, pcor_ref, W_ref, A_ref, probs_ref, state_ref):
    T = pk_ref.shape[0]
    blk = pk_ref.shape[1]
    Wp = W_ref[...]
    ramp = lax.broadcasted_iota(jnp.int32, (KC_PAD, blk), 0)

    def wsel(oh, c):
        return jnp.sum(oh * Wp[:, c:c + 1], axis=0, keepdims=True)

    state = jnp.broadcast_to(
        jax.nn.sigmoid(Wp[:, 4:5]), (KC_PAD, blk))
    for t in range(T):
        oc = (ramp == ck_ref[t:t + 1, :]).astype(jnp.float32)
        c2 = jax.nn.sigmoid(wsel(oc, 2))
        c3 = jax.nn.sigmoid(wsel(oc, 3))
        if t > 0:
            op = (ramp == pk_ref[t:t + 1, :]).astype(jnp.float32)
            p0 = jax.nn.sigmoid(wsel(op, 0))
            p1 = jax.nn.sigmoid(wsel(op, 1))
            p2 = jax.nn.sigmoid(wsel(op, 2))
            p3 = jax.nn.sigmoid(wsel(op, 3))
            pcor = pcor_ref[t:t + 1, :]
            ss = jnp.sum(state * op, axis=0, keepdims=True)
            po0 = jnp.where(pcor > 0.5, p2, 1.0 - p2)
            po1 = jnp.where(pcor > 0.5, p3, 1.0 - p3)
            filt = po1 * ss / (po0 * (1.0 - ss) + po1 * ss)
            pred = p0 * (1.0 - filt) + (1.0 - p1) * filt
            state = state * (1.0 - op) + op * pred
        cs = jnp.sum(state * oc, axis=0, keepdims=True)
        probs_ref[t:t + 1, :] = c2 * (1.0 - cs) + c3 * cs
    state_ref[...] = jax.lax.dot_general(
        A_ref[...], state, (((1,), (0,)), ((), ())),
        preferred_element_type=jnp.float32)


def kernel(prev_kc, curr_kc, prev_corr, A, W):
    B, T = prev_kc.shape
    V, K = A.shape
    Apad = jnp.pad(A, ((0, 0), (0, KC_PAD - K)))
    Wpad = jnp.pad(W, ((0, KC_PAD - K), (0, W_PAD - W.shape[1])))

    kc_of = _compute_kc_of(Apad)
    idx = jnp.concatenate(
        [prev_kc.reshape(-1), curr_kc.reshape(-1)]).astype(jnp.int32)
    ids = _sc_gather_ids(kc_of, idx)
    pkT = ids[:B * T].reshape(B, T).T
    ckT = ids[B * T:].reshape(B, T).T

    blk = 128
    probsT, stateT = pl.pallas_call(
        _bkt_body,
        grid=(B // blk,),
        in_specs=[
            pl.BlockSpec((T, blk), lambda i: (0, i)),
            pl.BlockSpec((T, blk), lambda i: (0, i)),
            pl.BlockSpec((T, blk), lambda i: (0, i)),
            pl.BlockSpec((KC_PAD, W_PAD), lambda i: (0, 0)),
            pl.BlockSpec((V, KC_PAD), lambda i: (0, 0)),
        ],
        out_specs=[
            pl.BlockSpec((T, blk), lambda i: (0, i)),
            pl.BlockSpec((V, blk), lambda i: (0, i)),
        ],
        out_shape=[
            jax.ShapeDtypeStruct((T, B), jnp.float32),
            jax.ShapeDtypeStruct((V, B), jnp.float32),
        ],
    )(pkT, ckT, prev_corr.T, Wpad, Apad)
    return probsT.T, stateT.T


# speedup vs baseline: 24.1789x; 1.0508x over previous
import functools

import jax
import jax.numpy as jnp
from jax import lax
from jax.experimental import pallas as pl
from jax.experimental.pallas import tpu as pltpu
from jax.experimental.pallas import tpu_sc as plsc

KC_PAD = 64
W_PAD = 8
L = 16


def _kc_of_body(A_ref, out_ref):
    a = A_ref[...]
    ramp = lax.broadcasted_iota(jnp.int32, a.shape, 1).astype(jnp.float32)
    out_ref[...] = jnp.sum(a * ramp, axis=1).astype(jnp.int32)


def _compute_kc_of(A):
    V = A.shape[0]
    return pl.pallas_call(
        _kc_of_body,
        out_shape=jax.ShapeDtypeStruct((V,), jnp.int32),
    )(A)


def _sc_gather_ids(table, idx):
    B = idx.shape[0]
    V = table.shape[0]
    info = plsc.get_sparse_core_info()
    nw = info.num_cores * info.num_subcores
    b_per_w = B // nw
    mesh = plsc.VectorSubcoreMesh(core_axis_name="c", subcore_axis_name="s")

    @functools.partial(
        pl.kernel,
        mesh=mesh,
        out_type=jax.ShapeDtypeStruct((B,), jnp.int32),
        compiler_params=pltpu.CompilerParams(needs_layout_passes=False),
        scratch_types=[
            pltpu.VMEM((V,), jnp.int32),
            pltpu.VMEM((b_per_w,), jnp.int32),
            pltpu.VMEM((b_per_w,), jnp.int32),
        ],
    )
    def k(table_hbm, idx_hbm, out_hbm, table_v, idx_v, out_v):
        wid = lax.axis_index("s") * info.num_cores + lax.axis_index("c")
        base = wid * b_per_w
        pltpu.sync_copy(table_hbm, table_v)
        pltpu.sync_copy(idx_hbm.at[pl.ds(base, b_per_w)], idx_v)
        for i in range(b_per_w // L):
            ivec = idx_v[pl.ds(i * L, L)]
            out_v[pl.ds(i * L, L)] = plsc.load_gather(table_v, [ivec])
        pltpu.sync_copy(out_v, out_hbm.at[pl.ds(base, b_per_w)])

    return k(table, idx)


def _bkt_body(pk_ref, ck_ref, pcor_ref, W_ref, A_ref, probs_ref, state_ref):
    T = pk_ref.shape[0]
    blk = pk_ref.shape[1]
    Wp = W_ref[...]
    ramp = lax.broadcasted_iota(jnp.int32, (KC_PAD, blk), 0)

    def wsel(oh, c):
        return jnp.sum(oh * Wp[:, c:c + 1], axis=0, keepdims=True)

    state = jnp.broadcast_to(
        jax.nn.sigmoid(Wp[:, 4:5]), (KC_PAD, blk))
    for t in range(T):
        oc = (ramp == ck_ref[t:t + 1, :]).astype(jnp.float32)
        c2 = jax.nn.sigmoid(wsel(oc, 2))
        c3 = jax.nn.sigmoid(wsel(oc, 3))
        if t > 0:
            op = (ramp == pk_ref[t:t + 1, :]).astype(jnp.float32)
            p0 = jax.nn.sigmoid(wsel(op, 0))
            p1 = jax.nn.sigmoid(wsel(op, 1))
            p2 = jax.nn.sigmoid(wsel(op, 2))
            p3 = jax.nn.sigmoid(wsel(op, 3))
            pcor = pcor_ref[t:t + 1, :]
            ss = jnp.sum(state * op, axis=0, keepdims=True)
            po0 = jnp.where(pcor > 0.5, p2, 1.0 - p2)
            po1 = jnp.where(pcor > 0.5, p3, 1.0 - p3)
            filt = po1 * ss / (po0 * (1.0 - ss) + po1 * ss)
            pred = p0 * (1.0 - filt) + (1.0 - p1) * filt
            state = state * (1.0 - op) + op * pred
        cs = jnp.sum(state * oc, axis=0, keepdims=True)
        probs_ref[t:t + 1, :] = c2 * (1.0 - cs) + c3 * cs
    state_ref[...] = jax.lax.dot_general(
        A_ref[...], state, (((1,), (0,)), ((), ())),
        preferred_element_type=jnp.float32)


def kernel(prev_kc, curr_kc, prev_corr, A, W):
    B, T = prev_kc.shape
    V, K = A.shape
    Apad = jnp.pad(A, ((0, 0), (0, KC_PAD - K)))
    Wpad = jnp.pad(W, ((0, KC_PAD - K), (0, W_PAD - W.shape[1])))

    kc_of = _compute_kc_of(A)
    idx = jnp.concatenate(
        [prev_kc.T.reshape(-1), curr_kc.T.reshape(-1)]).astype(jnp.int32)
    ids = _sc_gather_ids(kc_of, idx)
    pkT = ids[:B * T].reshape(T, B)
    ckT = ids[B * T:].reshape(T, B)

    blk = 128
    probsT, stateT = pl.pallas_call(
        _bkt_body,
        grid=(B // blk,),
        in_specs=[
            pl.BlockSpec((T, blk), lambda i: (0, i)),
            pl.BlockSpec((T, blk), lambda i: (0, i)),
            pl.BlockSpec((T, blk), lambda i: (0, i)),
            pl.BlockSpec((KC_PAD, W_PAD), lambda i: (0, 0)),
            pl.BlockSpec((V, KC_PAD), lambda i: (0, 0)),
        ],
        out_specs=[
            pl.BlockSpec((T, blk), lambda i: (0, i)),
            pl.BlockSpec((V, blk), lambda i: (0, i)),
        ],
        out_shape=[
            jax.ShapeDtypeStruct((T, B), jnp.float32),
            jax.ShapeDtypeStruct((V, B), jnp.float32),
        ],
    )(pkT, ckT, prev_corr.T, Wpad, Apad)
    return probsT.T, stateT.T


# speedup vs baseline: 30.0511x; 1.2429x over previous
import functools

import jax
import jax.numpy as jnp
from jax import lax
from jax.experimental import pallas as pl
from jax.experimental.pallas import tpu as pltpu
from jax.experimental.pallas import tpu_sc as plsc

L = 16
KC_BLK = 2048


def _kc_of_body(At_ref, out_ref):
    a = At_ref[...]
    ramp = lax.broadcasted_iota(jnp.int32, a.shape, 0).astype(jnp.float32)
    out_ref[...] = jnp.sum(a * ramp, axis=0).astype(jnp.int32)


def _compute_kc_of(At):
    K, V = At.shape
    grid = (pl.cdiv(V, KC_BLK),)
    return pl.pallas_call(
        _kc_of_body,
        grid=grid,
        in_specs=[pl.BlockSpec((K, KC_BLK), lambda i: (0, i))],
        out_specs=pl.BlockSpec((KC_BLK,), lambda i: (i,)),
        out_shape=jax.ShapeDtypeStruct((V,), jnp.int32),
    )(At)


def _sc_gather_ids(table, idx):
    N = idx.shape[0]
    V = table.shape[0]
    info = plsc.get_sparse_core_info()
    nw = info.num_cores * info.num_subcores
    n_per_w = N // nw
    mesh = plsc.VectorSubcoreMesh(core_axis_name="c", subcore_axis_name="s")

    @functools.partial(
        pl.kernel,
        mesh=mesh,
        out_type=jax.ShapeDtypeStruct((N,), jnp.int32),
        compiler_params=pltpu.CompilerParams(needs_layout_passes=False),
        scratch_types=[
            pltpu.VMEM((V,), jnp.int32),
            pltpu.VMEM((n_per_w,), jnp.int32),
            pltpu.VMEM((n_per_w,), jnp.int32),
        ],
    )
    def k(table_hbm, idx_hbm, out_hbm, table_v, idx_v, out_v):
        wid = lax.axis_index("s") * info.num_cores + lax.axis_index("c")
        base = wid * n_per_w
        pltpu.sync_copy(table_hbm, table_v)
        pltpu.sync_copy(idx_hbm.at[pl.ds(base, n_per_w)], idx_v)
        for i in range(n_per_w // L):
            ivec = idx_v[pl.ds(i * L, L)]
            out_v[pl.ds(i * L, L)] = plsc.load_gather(table_v, [ivec])
        pltpu.sync_copy(out_v, out_hbm.at[pl.ds(base, n_per_w)])

    return k(table, idx)


def _bkt_body(pk_ref, ck_ref, pcor_ref, W_ref, At_ref, probs_ref, state_ref):
    T = pk_ref.shape[0]
    blk = pk_ref.shape[1]
    K = W_ref.shape[0]
    Wp = W_ref[...]
    ramp = lax.broadcasted_iota(jnp.int32, (K, blk), 0)

    def wsel(oh, c):
        return jnp.sum(oh * Wp[:, c:c + 1], axis=0, keepdims=True)

    state = jnp.broadcast_to(jax.nn.sigmoid(Wp[:, 4:5]), (K, blk))
    for t in range(T):
        oc = (ramp == ck_ref[t:t + 1, :]).astype(jnp.float32)
        c2 = jax.nn.sigmoid(wsel(oc, 2))
        c3 = jax.nn.sigmoid(wsel(oc, 3))
        if t > 0:
            op = (ramp == pk_ref[t:t + 1, :]).astype(jnp.float32)
            p0 = jax.nn.sigmoid(wsel(op, 0))
            p1 = jax.nn.sigmoid(wsel(op, 1))
            p2 = jax.nn.sigmoid(wsel(op, 2))
            p3 = jax.nn.sigmoid(wsel(op, 3))
            pcor = pcor_ref[t:t + 1, :]
            ss = jnp.sum(state * op, axis=0, keepdims=True)
            po0 = jnp.where(pcor > 0.5, p2, 1.0 - p2)
            po1 = jnp.where(pcor > 0.5, p3, 1.0 - p3)
            filt = po1 * ss / (po0 * (1.0 - ss) + po1 * ss)
            pred = p0 * (1.0 - filt) + (1.0 - p1) * filt
            state = state * (1.0 - op) + op * pred
        cs = jnp.sum(state * oc, axis=0, keepdims=True)
        probs_ref[t:t + 1, :] = c2 * (1.0 - cs) + c3 * cs
    state_ref[...] = jax.lax.dot_general(
        At_ref[...], state, (((0,), (0,)), ((), ())),
        preferred_element_type=jnp.float32)


def kernel(prev_kc, curr_kc, prev_corr, A, W):
    B, T = prev_kc.shape
    V, K = A.shape
    At = A.T

    kc_of = _compute_kc_of(At)
    idx = jnp.concatenate(
        [prev_kc, curr_kc], axis=0).T.reshape(-1).astype(jnp.int32)
    ids2d = _sc_gather_ids(kc_of, idx).reshape(T, 2 * B)

    blk = 128
    nb = B // blk
    probsT, stateT = pl.pallas_call(
        _bkt_body,
        grid=(nb,),
        in_specs=[
            pl.BlockSpec((T, blk), lambda i: (0, i)),
            pl.BlockSpec((T, blk), lambda i: (0, i + nb)),
            pl.BlockSpec((T, blk), lambda i: (0, i)),
            pl.BlockSpec((K, 5), lambda i: (0, 0)),
            pl.BlockSpec((K, V), lambda i: (0, 0)),
        ],
        out_specs=[
            pl.BlockSpec((T, blk), lambda i: (0, i)),
            pl.BlockSpec((V, blk), lambda i: (0, i)),
        ],
        out_shape=[
            jax.ShapeDtypeStruct((T, B), jnp.float32),
            jax.ShapeDtypeStruct((V, B), jnp.float32),
        ],
        compiler_params=pltpu.CompilerParams(
            fuse_transposed_lhs_in_matmul=True),
    )(ids2d, ids2d, prev_corr.T, W, At)
    return probsT.T, stateT.T


# speedup vs baseline: 31.0933x; 1.0347x over previous
import functools

import jax
import jax.numpy as jnp
from jax import lax
from jax.experimental import pallas as pl
from jax.experimental.pallas import tpu as pltpu
from jax.experimental.pallas import tpu_sc as plsc

L = 16
KC_BLK = 2048


def _kc_of_body(At_ref, out_ref):
    a = At_ref[...]
    ramp = lax.broadcasted_iota(jnp.int32, a.shape, 0).astype(jnp.float32)
    out_ref[...] = jnp.sum(a * ramp, axis=0).astype(jnp.int32)


def _compute_kc_of(At):
    K, V = At.shape
    grid = (pl.cdiv(V, KC_BLK),)
    return pl.pallas_call(
        _kc_of_body,
        grid=grid,
        in_specs=[pl.BlockSpec((K, KC_BLK), lambda i: (0, i))],
        out_specs=pl.BlockSpec((KC_BLK,), lambda i: (i,)),
        out_shape=jax.ShapeDtypeStruct((V,), jnp.int32),
    )(At)


def _sc_gather_ids(table, idx):
    N = idx.shape[0]
    V = table.shape[0]
    info = plsc.get_sparse_core_info()
    nw = info.num_cores * info.num_subcores
    n_per_w = N // nw
    mesh = plsc.VectorSubcoreMesh(core_axis_name="c", subcore_axis_name="s")

    @functools.partial(
        pl.kernel,
        mesh=mesh,
        out_type=jax.ShapeDtypeStruct((N,), jnp.int32),
        compiler_params=pltpu.CompilerParams(needs_layout_passes=False),
        scratch_types=[
            pltpu.VMEM((V,), jnp.int32),
            pltpu.VMEM((n_per_w,), jnp.int32),
            pltpu.VMEM((n_per_w,), jnp.int32),
        ],
    )
    def k(table_hbm, idx_hbm, out_hbm, table_v, idx_v, out_v):
        wid = lax.axis_index("s") * info.num_cores + lax.axis_index("c")
        base = wid * n_per_w
        pltpu.sync_copy(table_hbm, table_v)
        pltpu.sync_copy(idx_hbm.at[pl.ds(base, n_per_w)], idx_v)
        for i in range(n_per_w // L):
            ivec = idx_v[pl.ds(i * L, L)]
            out_v[pl.ds(i * L, L)] = plsc.load_gather(table_v, [ivec])
        pltpu.sync_copy(out_v, out_hbm.at[pl.ds(base, n_per_w)])

    return k(table, idx)


def _bkt_body(ids_ref, pcor_ref, Wt_ref, At_ref, probs_ref, state_ref):
    T = pcor_ref.shape[0]
    blk = pcor_ref.shape[1]
    K = Wt_ref.shape[1]
    i = pl.program_id(0)
    B2 = ids_ref.shape[0] // T
    sw = jax.nn.sigmoid(Wt_ref[...].T)
    ramp = lax.broadcasted_iota(jnp.int32, (K, blk), 0)

    def row(base):
        return ids_ref[pl.ds(pl.multiple_of(base, blk), blk)].reshape(1, blk)

    def wsel(oh, c):
        return jnp.sum(oh * sw[:, c:c + 1], axis=0, keepdims=True)

    state = jnp.broadcast_to(sw[:, 4:5], (K, blk))
    for t in range(T):
        oc = (ramp == row(t * B2 + B2 // 2 + i * blk)).astype(jnp.float32)
        c2 = wsel(oc, 2)
        c3 = wsel(oc, 3)
        if t > 0:
            op = (ramp == row(t * B2 + i * blk)).astype(jnp.float32)
            p0 = wsel(op, 0)
            p1 = wsel(op, 1)
            p2 = wsel(op, 2)
            p3 = wsel(op, 3)
            pcor = pcor_ref[t:t + 1, :]
            ss = jnp.sum(state * op, axis=0, keepdims=True)
            po0 = jnp.where(pcor > 0.5, p2, 1.0 - p2)
            po1 = jnp.where(pcor > 0.5, p3, 1.0 - p3)
            filt = po1 * ss / (po0 * (1.0 - ss) + po1 * ss)
            pred = p0 * (1.0 - filt) + (1.0 - p1) * filt
            state = state * (1.0 - op) + op * pred
        cs = jnp.sum(state * oc, axis=0, keepdims=True)
        probs_ref[t:t + 1, :] = c2 * (1.0 - cs) + c3 * cs
    state_ref[...] = jax.lax.dot_general(
        At_ref[...], state, (((0,), (0,)), ((), ())),
        preferred_element_type=jnp.float32)


def kernel(prev_kc, curr_kc, prev_corr, A, W):
    B, T = prev_kc.shape
    V, K = A.shape
    At = A.T

    kc_of = _compute_kc_of(At)
    idx = jnp.concatenate(
        [prev_kc, curr_kc], axis=0).T.reshape(-1).astype(jnp.int32)
    ids = _sc_gather_ids(kc_of, idx)

    blk = 128
    probsT, stateT = pl.pallas_call(
        _bkt_body,
        grid=(B // blk,),
        in_specs=[
            pl.BlockSpec((2 * B * T,), lambda i: (0,)),
            pl.BlockSpec((T, blk), lambda i: (0, i)),
            pl.BlockSpec((5, K), lambda i: (0, 0)),
            pl.BlockSpec((K, V), lambda i: (0, 0)),
        ],
        out_specs=[
            pl.BlockSpec((T, blk), lambda i: (0, i)),
            pl.BlockSpec((V, blk), lambda i: (0, i)),
        ],
        out_shape=[
            jax.ShapeDtypeStruct((T, B), jnp.float32),
            jax.ShapeDtypeStruct((V, B), jnp.float32),
        ],
        compiler_params=pltpu.CompilerParams(
            fuse_transposed_lhs_in_matmul=True),
    )(ids, prev_corr.T, W.T, At)
    return probsT.T, stateT.T


# speedup vs baseline: 31.1273x; 1.0011x over previous
import functools

import jax
import jax.numpy as jnp
from jax import lax
from jax.experimental import pallas as pl
from jax.experimental.pallas import tpu as pltpu
from jax.experimental.pallas import tpu_sc as plsc

L = 16
KC_BLK = 2048


def _kc_of_body(At_ref, out_ref):
    a = At_ref[...]
    ramp = lax.broadcasted_iota(jnp.int32, a.shape, 0).astype(jnp.float32)
    out_ref[...] = jnp.sum(a * ramp, axis=0).astype(jnp.int32)


def _compute_kc_of(At):
    K, V = At.shape
    grid = (pl.cdiv(V, KC_BLK),)
    return pl.pallas_call(
        _kc_of_body,
        grid=grid,
        in_specs=[pl.BlockSpec((K, KC_BLK), lambda i: (0, i))],
        out_specs=pl.BlockSpec((KC_BLK,), lambda i: (i,)),
        out_shape=jax.ShapeDtypeStruct((V,), jnp.int32),
    )(At)


def _sc_gather_ids(table, idx):
    N = idx.shape[0]
    V = table.shape[0]
    info = plsc.get_sparse_core_info()
    nw = info.num_cores * info.num_subcores
    n_per_w = N // nw
    mesh = plsc.VectorSubcoreMesh(core_axis_name="c", subcore_axis_name="s")

    @functools.partial(
        pl.kernel,
        mesh=mesh,
        out_type=jax.ShapeDtypeStruct((N,), jnp.int32),
        compiler_params=pltpu.CompilerParams(needs_layout_passes=False),
        scratch_types=[
            pltpu.VMEM((V,), jnp.int32),
            pltpu.VMEM((n_per_w,), jnp.int32),
            pltpu.VMEM((n_per_w,), jnp.int32),
        ],
    )
    def k(table_hbm, idx_hbm, out_hbm, table_v, idx_v, out_v):
        wid = lax.axis_index("s") * info.num_cores + lax.axis_index("c")
        base = wid * n_per_w
        pltpu.sync_copy(table_hbm, table_v)
        pltpu.sync_copy(idx_hbm.at[pl.ds(base, n_per_w)], idx_v)
        for i in range(n_per_w // L):
            ivec = idx_v[pl.ds(i * L, L)]
            out_v[pl.ds(i * L, L)] = plsc.load_gather(table_v, [ivec])
        pltpu.sync_copy(out_v, out_hbm.at[pl.ds(base, n_per_w)])

    return k(table, idx)


def _bkt_body(ids_ref, pcor_ref, Wt_ref, At_ref, probs_ref, state_ref):
    T = pcor_ref.shape[0]
    blk = pcor_ref.shape[1]
    K = Wt_ref.shape[1]
    i = pl.program_id(0)
    B2 = ids_ref.shape[0] // T
    sw = jax.nn.sigmoid(Wt_ref[...].T)
    ramp = lax.broadcasted_iota(jnp.int32, (K, blk), 0)

    def row(base):
        return ids_ref[pl.ds(pl.multiple_of(base, blk), blk)].reshape(1, blk)

    def wsel(oh, c):
        return jnp.sum(oh * sw[:, c:c + 1], axis=0, keepdims=True)

    state = jnp.broadcast_to(sw[:, 4:5], (K, blk))
    for t in range(T):
        oc = (ramp == row(t * B2 + B2 // 2 + i * blk)).astype(jnp.float32)
        c2 = wsel(oc, 2)
        c3 = wsel(oc, 3)
        if t > 0:
            opb = ramp == row(t * B2 + i * blk)
            op = opb.astype(jnp.float32)
            p0 = wsel(op, 0)
            p1 = wsel(op, 1)
            p2 = wsel(op, 2)
            p3 = wsel(op, 3)
            pcor = pcor_ref[t:t + 1, :]
            ss = jnp.sum(state * op, axis=0, keepdims=True)
            po0 = jnp.where(pcor > 0.5, p2, 1.0 - p2)
            po1 = jnp.where(pcor > 0.5, p3, 1.0 - p3)
            filt = po1 * ss / (po0 * (1.0 - ss) + po1 * ss)
            pred = p0 * (1.0 - filt) + (1.0 - p1) * filt
            state = jnp.where(opb, pred, state)
        cs = jnp.sum(state * oc, axis=0, keepdims=True)
        probs_ref[t:t + 1, :] = c2 * (1.0 - cs) + c3 * cs
    state_ref[...] = jax.lax.dot_general(
        At_ref[...], state, (((0,), (0,)), ((), ())),
        preferred_element_type=jnp.float32)


def kernel(prev_kc, curr_kc, prev_corr, A, W):
    B, T = prev_kc.shape
    V, K = A.shape
    At = A.T

    kc_of = _compute_kc_of(At)
    idx = jnp.concatenate(
        [prev_kc, curr_kc], axis=0).T.reshape(-1).astype(jnp.int32)
    ids = _sc_gather_ids(kc_of, idx)

    blk = 128
    probsT, stateT = pl.pallas_call(
        _bkt_body,
        grid=(B // blk,),
        in_specs=[
            pl.BlockSpec((2 * B * T,), lambda i: (0,)),
            pl.BlockSpec((T, blk), lambda i: (0, i)),
            pl.BlockSpec((5, K), lambda i: (0, 0)),
            pl.BlockSpec((K, V), lambda i: (0, 0)),
        ],
        out_specs=[
            pl.BlockSpec((T, blk), lambda i: (0, i)),
            pl.BlockSpec((V, blk), lambda i: (0, i)),
        ],
        out_shape=[
            jax.ShapeDtypeStruct((T, B), jnp.float32),
            jax.ShapeDtypeStruct((V, B), jnp.float32),
        ],
        compiler_params=pltpu.CompilerParams(
            fuse_transposed_lhs_in_matmul=True),
    )(ids, prev_corr.T, W.T, At)
    return probsT.T, stateT.T


# speedup vs baseline: 32.3264x; 1.0385x over previous
import functools

import jax
import jax.numpy as jnp
from jax import lax
from jax.experimental import pallas as pl
from jax.experimental.pallas import tpu as pltpu
from jax.experimental.pallas import tpu_sc as plsc

L = 16
KC_BLK = 2048


def _kc_of_body(At_ref, out_ref):
    a = At_ref[...]
    ramp = lax.broadcasted_iota(jnp.int32, a.shape, 0).astype(jnp.float32)
    out_ref[...] = jnp.sum(a * ramp, axis=0).astype(jnp.int32)


def _compute_kc_of(At):
    K, V = At.shape
    grid = (pl.cdiv(V, KC_BLK),)
    return pl.pallas_call(
        _kc_of_body,
        grid=grid,
        in_specs=[pl.BlockSpec((K, KC_BLK), lambda i: (0, i))],
        out_specs=pl.BlockSpec((KC_BLK,), lambda i: (i,)),
        out_shape=jax.ShapeDtypeStruct((V,), jnp.int32),
    )(At)


def _sc_gather_ids(table, idx):
    N = idx.shape[0]
    V = table.shape[0]
    info = plsc.get_sparse_core_info()
    nw = info.num_cores * info.num_subcores
    n_per_w = N // nw
    mesh = plsc.VectorSubcoreMesh(core_axis_name="c", subcore_axis_name="s")

    @functools.partial(
        pl.kernel,
        mesh=mesh,
        out_type=jax.ShapeDtypeStruct((N,), jnp.int32),
        compiler_params=pltpu.CompilerParams(needs_layout_passes=False),
        scratch_types=[
            pltpu.VMEM((V,), jnp.int32),
            pltpu.VMEM((n_per_w,), jnp.int32),
            pltpu.VMEM((n_per_w,), jnp.int32),
        ],
    )
    def k(table_hbm, idx_hbm, out_hbm, table_v, idx_v, out_v):
        wid = lax.axis_index("s") * info.num_cores + lax.axis_index("c")
        base = wid * n_per_w
        pltpu.sync_copy(table_hbm, table_v)
        pltpu.sync_copy(idx_hbm.at[pl.ds(base, n_per_w)], idx_v)
        for i in range(n_per_w // L):
            ivec = idx_v[pl.ds(i * L, L)]
            out_v[pl.ds(i * L, L)] = plsc.load_gather(table_v, [ivec])
        pltpu.sync_copy(out_v, out_hbm.at[pl.ds(base, n_per_w)])

    return k(table, idx)


def _bkt_body(ids_ref, pcor_ref, Wt_ref, At_ref, probs_ref, state_ref):
    T = pcor_ref.shape[0]
    blk = pcor_ref.shape[1]
    K = Wt_ref.shape[1]
    i = pl.program_id(0)
    B2 = ids_ref.shape[0] // T
    sw = jax.nn.sigmoid(Wt_ref[...].T)
    ramp = lax.broadcasted_iota(jnp.int32, (K, blk), 0)

    def row(base):
        return ids_ref[pl.ds(pl.multiple_of(base, blk), blk)].reshape(1, blk)

    def wsel(oh, c):
        return jnp.sum(oh * sw[:, c:c + 1], axis=0, keepdims=True)

    state = jnp.broadcast_to(sw[:, 4:5], (K, blk))
    for t in range(T):
        oc = (ramp == row(t * B2 + B2 // 2 + i * blk)).astype(jnp.float32)
        c2 = wsel(oc, 2)
        c3 = wsel(oc, 3)
        if t > 0:
            opb = ramp == row(t * B2 + i * blk)
            op = opb.astype(jnp.float32)
            p0 = wsel(op, 0)
            p1 = wsel(op, 1)
            p2 = wsel(op, 2)
            p3 = wsel(op, 3)
            pcor = pcor_ref[t:t + 1, :]
            ss = jnp.sum(state * op, axis=0, keepdims=True)
            po0 = jnp.where(pcor > 0.5, p2, 1.0 - p2)
            po1 = jnp.where(pcor > 0.5, p3, 1.0 - p3)
            filt = po1 * ss / (po0 * (1.0 - ss) + po1 * ss)
            pred = p0 * (1.0 - filt) + (1.0 - p1) * filt
            state = jnp.where(opb, pred, state)
        cs = jnp.sum(state * oc, axis=0, keepdims=True)
        probs_ref[t:t + 1, :] = c2 * (1.0 - cs) + c3 * cs
    state_ref[...] = jax.lax.dot_general(
        At_ref[...], state, (((0,), (0,)), ((), ())),
        preferred_element_type=jnp.float32)


def kernel(prev_kc, curr_kc, prev_corr, A, W):
    B, T = prev_kc.shape
    V, K = A.shape
    At = A.T

    kc_of = _compute_kc_of(At)
    idx = jnp.concatenate(
        [prev_kc, curr_kc], axis=0).T.reshape(-1).astype(jnp.int32)
    ids = _sc_gather_ids(kc_of, idx)

    blk = 256
    probsT, stateT = pl.pallas_call(
        _bkt_body,
        grid=(B // blk,),
        in_specs=[
            pl.BlockSpec((2 * B * T,), lambda i: (0,)),
            pl.BlockSpec((T, blk), lambda i: (0, i)),
            pl.BlockSpec((5, K), lambda i: (0, 0)),
            pl.BlockSpec((K, V), lambda i: (0, 0)),
        ],
        out_specs=[
            pl.BlockSpec((T, blk), lambda i: (0, i)),
            pl.BlockSpec((V, blk), lambda i: (0, i)),
        ],
        out_shape=[
            jax.ShapeDtypeStruct((T, B), jnp.float32),
            jax.ShapeDtypeStruct((V, B), jnp.float32),
        ],
        compiler_params=pltpu.CompilerParams(
            fuse_transposed_lhs_in_matmul=True),
    )(ids, prev_corr.T, W.T, At)
    return probsT.T, stateT.T
